# Initial kernel scaffold; baseline (speedup 1.0000x reference)
import jax
import jax.numpy as jnp
from jax.experimental import pallas as pl


def kernel(x, edge_index, W1l, W1r, b1, g1, be1, W2l, W2r, b2, g2, be2):
    raise NotImplementedError("write your pallas kernel here")


# speedup vs baseline: 3.9024x; 3.9024x over previous
import functools

import jax
import jax.numpy as jnp
from jax import lax
from jax.experimental import pallas as pl
from jax.experimental.pallas import tpu as pltpu
from jax.experimental.pallas import tpu_sc as plsc

N = 10000
E = 320000
D = 128
EPS = 1e-5

NC = 2
NS = 16
NW = NC * NS

K = 64
CHUNKS_PER_W = 160
E_PAD = NW * K * CHUNKS_PER_W
N_PAD = 10240
ROWS_PER_TILE = N_PAD // NS
PAD_DST = N + 16


def _make_agg(with_counts: bool):
    mesh = plsc.VectorSubcoreMesh(core_axis_name="c", subcore_axis_name="s",
                                  num_cores=NC, num_subcores=NS)
    out_type = [jax.ShapeDtypeStruct((NC, N_PAD, D), jnp.float32)]
    if with_counts:
        out_type.append(jax.ShapeDtypeStruct((NC, N_PAD), jnp.float32))

    def body(table_hbm, src_hbm, dst_hbm, *rest):
        if with_counts:
            (acc_out, cnt_out, src_v, dst_v, rows_v, ones_v,
             acc_sh, cnt_sh, sem0, sem1) = rest
        else:
            (acc_out, src_v, dst_v, rows_v, acc_sh, sem0, sem1) = rest
        cid = lax.axis_index("c")
        sid = lax.axis_index("s")
        wid = sid * NC + cid
        sems = (sem0, sem1)

        def zrow(r, _):
            for c8 in range(D // 16):
                rows_v[0, r, pl.ds(c8 * 16, 16)] = jnp.zeros((16,), jnp.float32)
            return _
        lax.fori_loop(0, K, zrow, None)
        if with_counts:
            for c8 in range(K // 16):
                ones_v[pl.ds(c8 * 16, 16)] = jnp.ones((16,), jnp.float32)

        for k in range(ROWS_PER_TILE // K):
            pltpu.sync_copy(rows_v.at[0],
                            acc_sh.at[pl.ds(sid * ROWS_PER_TILE + k * K, K)])
        if with_counts:
            for k in range(ROWS_PER_TILE // D):
                pltpu.sync_copy(rows_v.at[0, 0],
                                cnt_sh.at[pl.ds(sid * ROWS_PER_TILE + k * D, D)])
        plsc.subcore_barrier()

        def gather(c, b):
            pltpu.async_copy(table_hbm.at[src_v.at[c]], rows_v.at[b], sems[b])

        def gwait(b):
            pltpu.make_async_copy(table_hbm.at[src_v.at[0]], rows_v.at[b],
                                  sems[b]).wait()

        def scatter(c, b):
            pltpu.sync_copy(rows_v.at[b], acc_sh.at[dst_v.at[c]], add=True)
            if with_counts:
                pltpu.sync_copy(ones_v, cnt_sh.at[dst_v.at[c]], add=True)

        CH = CHUNKS_PER_W // 2
        for half in range(2):
            row0 = wid * CHUNKS_PER_W + half * CH
            pltpu.sync_copy(src_hbm.at[pl.ds(row0, CH)], src_v)
            pltpu.sync_copy(dst_hbm.at[pl.ds(row0, CH)], dst_v)

            gather(0, 0)

            def step(t, _):
                c0 = 2 * t
                gather(c0 + 1, 1)
                gwait(0)
                scatter(c0, 0)
                gather(c0 + 2, 0)
                gwait(1)
                scatter(c0 + 1, 1)
                return _
            lax.fori_loop(0, (CH - 2) // 2, step, None)

            gather(CH - 1, 1)
            gwait(0)
            scatter(CH - 2, 0)
            gwait(1)
            scatter(CH - 1, 1)
        plsc.subcore_barrier()

        pltpu.sync_copy(acc_sh.at[pl.ds(sid * ROWS_PER_TILE, ROWS_PER_TILE)],
                        acc_out.at[cid, pl.ds(sid * ROWS_PER_TILE, ROWS_PER_TILE)])
        if with_counts:
            pltpu.sync_copy(cnt_sh.at[pl.ds(sid * ROWS_PER_TILE, ROWS_PER_TILE)],
                            cnt_out.at[cid, pl.ds(sid * ROWS_PER_TILE, ROWS_PER_TILE)])

    scratch = [
        pltpu.VMEM((CHUNKS_PER_W // 2, K), jnp.int32),
        pltpu.VMEM((CHUNKS_PER_W // 2, K), jnp.int32),
        pltpu.VMEM((2, K, D), jnp.float32),
    ]
    if with_counts:
        scratch += [
            pltpu.VMEM((K,), jnp.float32),
            pltpu.VMEM_SHARED((N_PAD, D), jnp.float32),
            pltpu.VMEM_SHARED((N_PAD,), jnp.float32),
        ]
    else:
        scratch += [
            pltpu.VMEM_SHARED((N_PAD, D), jnp.float32),
        ]
    scratch += [pltpu.SemaphoreType.DMA, pltpu.SemaphoreType.DMA]

    return pl.kernel(body, out_type=tuple(out_type), mesh=mesh,
                     scratch_types=tuple(scratch))


@functools.lru_cache(maxsize=None)
def _get_agg(with_counts: bool):
    return _make_agg(with_counts)


def _tc_layer(residual: bool):
    def body(x_ref, parts_ref, cnt_ref, wl_ref, wr_ref, b_ref, g_ref,
             be_ref, *rest):
        if residual:
            res_ref, out_ref = rest
        else:
            (out_ref,) = rest
        s = parts_ref[0, pl.ds(0, N), :] + parts_ref[1, pl.ds(0, N), :]
        c = cnt_ref[0, pl.ds(0, N), :] + cnt_ref[1, pl.ds(0, N), :]
        mean = s * (1.0 / jnp.maximum(c, 1.0))
        h = (jnp.dot(mean, wl_ref[...], preferred_element_type=jnp.float32)
             + jnp.dot(x_ref[...], wr_ref[...], preferred_element_type=jnp.float32)
             + b_ref[...])
        mu = jnp.mean(h, axis=0, keepdims=True)
        var = jnp.mean((h - mu) ** 2, axis=0, keepdims=True)
        hn = g_ref[...] * (h - mu) * lax.rsqrt(var + EPS) + be_ref[...]
        if residual:
            hn = hn + res_ref[...]
        out_ref[...] = jnp.maximum(hn, 0.0)

    return pl.pallas_call(
        body, out_shape=jax.ShapeDtypeStruct((N, D), jnp.float32))


_tc1 = _tc_layer(False)
_tc2 = _tc_layer(True)


def kernel(x, edge_index, W1l, W1r, b1, g1, be1, W2l, W2r, b2, g2, be2):
    ei = edge_index.astype(jnp.int32)
    pad = E_PAD - E
    src = jnp.concatenate([ei[0], jnp.zeros((pad,), jnp.int32)]).reshape(-1, K)
    dst = jnp.concatenate([ei[1], jnp.full((pad,), PAD_DST, jnp.int32)]).reshape(-1, K)

    parts1, cnts = _get_agg(True)(x, src, dst)
    cnt3 = cnts.reshape(NC, N_PAD, 1)
    h1 = _tc1(x, parts1, cnt3, W1l.T, W1r.T, b1.reshape(1, D),
              g1.reshape(1, D), be1.reshape(1, D))
    (parts2,) = _get_agg(False)(h1, src, dst)
    out = _tc2(h1, parts2, cnt3, W2l.T, W2r.T, b2.reshape(1, D),
               g2.reshape(1, D), be2.reshape(1, D), x)
    return out
